# baseline (speedup 1.0000x reference)
import functools
import jax
import jax.numpy as jnp
from jax import lax
from jax.experimental import pallas as pl
from jax.experimental.pallas import tpu as pltpu
from jax.experimental.pallas import tpu_sc as plsc

NC, NS, LANES = 2, 16, 16
NW = NC * NS
K = 128
ZR = 64
BLK = 2048

FRAC_DEG = 0.55
FRAC_S64 = 0.55
FRAC_S16 = 0.56


def _split(total, frac):
    a = int(round(total * frac / 8.0)) * 8
    a = max(8, min(total - 8, a))
    return a, total - a


def _sc_mesh():
    return plsc.VectorSubcoreMesh(core_axis_name="c", subcore_axis_name="s")


def _deg_kernel(n_pad, cha, chb):
    rpt = n_pad // NS
    chm = max(cha, chb)

    def body(dst_hbm, out_hbm, idx_d, zbuf, obuf, deg_sh):
        c = lax.axis_index("c")
        s = lax.axis_index("s")

        def zb(i, _):
            zbuf[pl.ds(i * LANES, LANES)] = jnp.zeros((LANES,), jnp.float32)
            return _
        lax.fori_loop(0, rpt // LANES, zb, None)

        def ob(i, _):
            obuf[pl.ds(i * LANES, LANES)] = jnp.ones((LANES,), jnp.float32)
            return _
        lax.fori_loop(0, K // LANES, ob, None)

        pltpu.sync_copy(zbuf, deg_sh.at[pl.ds(s * rpt, rpt)])

        @pl.when(c == 0)
        def _():
            pltpu.sync_copy(dst_hbm.at[pl.ds(s * cha, cha)],
                            idx_d.at[pl.ds(0, cha)])

        @pl.when(c == 1)
        def _():
            pltpu.sync_copy(dst_hbm.at[pl.ds(NS * cha + s * chb, chb)],
                            idx_d.at[pl.ds(0, chb)])

        plsc.subcore_barrier()

        def step(j, _):
            pltpu.sync_copy(obuf, deg_sh.at[idx_d.at[j]], add=True)
            return _

        @pl.when(c == 0)
        def _():
            lax.fori_loop(0, cha, step, None)

        @pl.when(c == 1)
        def _():
            lax.fori_loop(0, chb, step, None)

        plsc.subcore_barrier()
        pltpu.sync_copy(deg_sh.at[pl.ds(s * rpt, rpt)],
                        out_hbm.at[c, pl.ds(s * rpt, rpt)])

    return pl.kernel(
        body,
        out_type=jax.ShapeDtypeStruct((NC, n_pad), jnp.float32),
        mesh=_sc_mesh(),
        scratch_types=[
            pltpu.VMEM((chm, K), jnp.int32),
            pltpu.VMEM((rpt,), jnp.float32),
            pltpu.VMEM((K,), jnp.float32),
            pltpu.VMEM_SHARED((n_pad,), jnp.float32),
        ],
        compiler_params=pltpu.CompilerParams(use_tc_tiling_on_sc=False),
    )


def _segsum_kernel(n_pad, d, cha, chb):
    rpt = n_pad // NS
    chm = max(cha, chb)

    def body(hs_hbm, src_hbm, dst_hbm, out_hbm, idx_s, idx_d, r0, r1,
             zbuf, agg_sh, hs_sh, g0, g1, c0, c1):
        rows = [r0, r1]
        gsem = [g0, g1]
        csem = [c0, c1]
        c = lax.axis_index("c")
        s = lax.axis_index("s")

        pltpu.sync_copy(hs_hbm.at[pl.ds(s * rpt, rpt)],
                        hs_sh.at[pl.ds(s * rpt, rpt)])

        def zb(i, _):
            zbuf[i // (d // LANES), pl.ds((i % (d // LANES)) * LANES, LANES)] = (
                jnp.zeros((LANES,), jnp.float32))
            return _
        lax.fori_loop(0, ZR * d // LANES, zb, None)

        def zs(r, _):
            pltpu.sync_copy(zbuf, agg_sh.at[pl.ds(s * rpt + r * ZR, ZR)])
            return _
        lax.fori_loop(0, rpt // ZR, zs, None)

        @pl.when(c == 0)
        def _():
            pltpu.sync_copy(src_hbm.at[pl.ds(s * cha, cha)],
                            idx_s.at[pl.ds(0, cha)])
            pltpu.sync_copy(dst_hbm.at[pl.ds(s * cha, cha)],
                            idx_d.at[pl.ds(0, cha)])

        @pl.when(c == 1)
        def _():
            pltpu.sync_copy(src_hbm.at[pl.ds(NS * cha + s * chb, chb)],
                            idx_s.at[pl.ds(0, chb)])
            pltpu.sync_copy(dst_hbm.at[pl.ds(NS * cha + s * chb, chb)],
                            idx_d.at[pl.ds(0, chb)])

        plsc.subcore_barrier()

        def run_edges(chc):
            for b in range(2):
                pltpu.async_copy(hs_sh.at[idx_s.at[b]], rows[b], gsem[b])

            def step(j2, _):
                j = j2 * 2
                for b in range(2):
                    pltpu.make_async_copy(hs_sh.at[idx_s.at[0]], rows[b],
                                          gsem[b]).wait()
                    pltpu.async_copy(rows[b], agg_sh.at[idx_d.at[j + b]],
                                     csem[b], add=True)
                for b in range(2):
                    pltpu.make_async_copy(rows[b], agg_sh.at[idx_d.at[0]],
                                          csem[b]).wait()
                    jn = jnp.minimum(j + 2 + b, chc - 1)
                    pltpu.async_copy(hs_sh.at[idx_s.at[jn]], rows[b], gsem[b])
                return _
            lax.fori_loop(0, chc // 2, step, None)
            for b in range(2):
                pltpu.make_async_copy(hs_sh.at[idx_s.at[0]], rows[b],
                                      gsem[b]).wait()

        @pl.when(c == 0)
        def _():
            run_edges(cha)

        @pl.when(c == 1)
        def _():
            run_edges(chb)

        plsc.subcore_barrier()
        pltpu.sync_copy(agg_sh.at[pl.ds(s * rpt, rpt)],
                        out_hbm.at[c, pl.ds(s * rpt, rpt)])

    return pl.kernel(
        body,
        out_type=jax.ShapeDtypeStruct((NC, n_pad, d), jnp.float32),
        mesh=_sc_mesh(),
        scratch_types=[
            pltpu.VMEM((chm, K), jnp.int32),
            pltpu.VMEM((chm, K), jnp.int32),
            pltpu.VMEM((K, d), jnp.float32),
            pltpu.VMEM((K, d), jnp.float32),
            pltpu.VMEM((ZR, d), jnp.float32),
            pltpu.VMEM_SHARED((n_pad, d), jnp.float32),
            pltpu.VMEM_SHARED((n_pad, d), jnp.float32),
        ] + [pltpu.SemaphoreType.DMA] * 4,
        compiler_params=pltpu.CompilerParams(use_tc_tiling_on_sc=False),
    )


def _tc1(n, n_pad, f, h):
    def body(degT_ref, x_ref, w_ref, b_ref, hs_ref, isq_ref):
        i = pl.program_id(0)
        deg = degT_ref[:, 0:1] + degT_ref[:, 1:2] + 1.0
        isq = lax.rsqrt(deg)
        rows = i * BLK + lax.broadcasted_iota(jnp.int32, (BLK, 1), 0)
        isq = jnp.where(rows < n, isq, 0.0)
        hw = jnp.dot(x_ref[...], w_ref[...],
                     preferred_element_type=jnp.float32) + b_ref[...]
        hs_ref[...] = hw * isq
        isq_ref[...] = isq

    grid = n_pad // BLK
    return pl.pallas_call(
        body,
        grid=(grid,),
        in_specs=[
            pl.BlockSpec((BLK, 2), lambda i: (i, 0)),
            pl.BlockSpec((BLK, f), lambda i: (i, 0)),
            pl.BlockSpec((f, h), lambda i: (0, 0)),
            pl.BlockSpec((1, h), lambda i: (0, 0)),
        ],
        out_specs=[
            pl.BlockSpec((BLK, h), lambda i: (i, 0)),
            pl.BlockSpec((BLK, 1), lambda i: (i, 0)),
        ],
        out_shape=[
            jax.ShapeDtypeStruct((n_pad, h), jnp.float32),
            jax.ShapeDtypeStruct((n_pad, 1), jnp.float32),
        ],
    )


def _tc2(n_pad, h, d2):
    def body(s1_ref, hs1_ref, isq_ref, w_ref, b_ref, hs2_ref):
        isq = isq_ref[...]
        ssum = s1_ref[0] + s1_ref[1] + hs1_ref[...]
        embed = jnp.maximum(isq * ssum, 0.0)
        hw = jnp.dot(embed, w_ref[...],
                     preferred_element_type=jnp.float32) + b_ref[...]
        hs2_ref[...] = hw * isq

    grid = n_pad // BLK
    return pl.pallas_call(
        body,
        grid=(grid,),
        in_specs=[
            pl.BlockSpec((NC, BLK, h), lambda i: (0, i, 0)),
            pl.BlockSpec((BLK, h), lambda i: (i, 0)),
            pl.BlockSpec((BLK, 1), lambda i: (i, 0)),
            pl.BlockSpec((h, d2), lambda i: (0, 0)),
            pl.BlockSpec((1, d2), lambda i: (0, 0)),
        ],
        out_specs=pl.BlockSpec((BLK, d2), lambda i: (i, 0)),
        out_shape=jax.ShapeDtypeStruct((n_pad, d2), jnp.float32),
    )


def _tc3(n_pad, d2):
    def body(s2_ref, hs2_ref, isq_ref, out_ref):
        out_ref[...] = isq_ref[...] * (s2_ref[0] + s2_ref[1] + hs2_ref[...])

    grid = n_pad // BLK
    return pl.pallas_call(
        body,
        grid=(grid,),
        in_specs=[
            pl.BlockSpec((NC, BLK, d2), lambda i: (0, i, 0)),
            pl.BlockSpec((BLK, d2), lambda i: (i, 0)),
            pl.BlockSpec((BLK, 1), lambda i: (i, 0)),
        ],
        out_specs=pl.BlockSpec((BLK, d2), lambda i: (i, 0)),
        out_shape=jax.ShapeDtypeStruct((n_pad, d2), jnp.float32),
    )


def kernel(x, y, edge_index, W1, b1, W2, b2):
    n, f = x.shape
    h = W1.shape[1]
    c_out = W2.shape[1]
    d2 = 16

    src = edge_index[0]
    dst = edge_index[1]
    e = src.shape[0]
    total_ch = -(-(-(-e // (NS * K))) // 16) * 16
    e_pad = NS * K * total_ch
    n_pad = -(-(n + 1) // (NS * ZR)) * (NS * ZR)

    pad_e = e_pad - e
    srcp = jnp.concatenate(
        [src, jnp.full((pad_e,), n, jnp.int32)]).reshape(NS * total_ch, K)
    dstp = jnp.concatenate(
        [dst, jnp.full((pad_e,), n, jnp.int32)]).reshape(NS * total_ch, K)
    w2p = jnp.pad(W2, ((0, 0), (0, d2 - c_out)))
    b1r = b1.reshape(1, h)
    b2p = jnp.pad(b2, (0, d2 - c_out)).reshape(1, d2)

    deg_parts = _deg_kernel(n_pad, *_split(total_ch, FRAC_DEG))(dstp)
    hs1, isq = _tc1(n, n_pad, f, h)(deg_parts.T, x, W1, b1r)
    s1 = _segsum_kernel(n_pad, h, *_split(total_ch, FRAC_S64))(hs1, srcp, dstp)
    hs2 = _tc2(n_pad, h, d2)(s1, hs1, isq, w2p, b2p)
    s2 = _segsum_kernel(n_pad, d2, *_split(total_ch, FRAC_S16))(hs2, srcp, dstp)
    logits_p = _tc3(n_pad, d2)(s2, hs2, isq)
    return logits_p[:n, :c_out]


# speedup vs baseline: 1.0893x; 1.0893x over previous
import functools
import jax
import jax.numpy as jnp
from jax import lax
from jax.experimental import pallas as pl
from jax.experimental.pallas import tpu as pltpu
from jax.experimental.pallas import tpu_sc as plsc

NC, NS, LANES = 2, 16, 16
NW = NC * NS
K = 128
ZR = 64
BLK = 2048

FRAC_DEG = 0.55
FRAC_S64 = 0.55
FRAC_S16 = 0.56


def _split(total, frac):
    a = int(round(total * frac / 8.0)) * 8
    a = max(8, min(total - 8, a))
    return a, total - a


def _sc_mesh():
    return plsc.VectorSubcoreMesh(core_axis_name="c", subcore_axis_name="s")


def _deg_kernel(n_pad, cha, chb):
    rpt = n_pad // NS
    chm = max(cha, chb)

    def body(ei_hbm, out_hbm, idx_d, zbuf, obuf, deg_sh):
        c = lax.axis_index("c")
        s = lax.axis_index("s")

        def zb(i, _):
            zbuf[pl.ds(i * LANES, LANES)] = jnp.zeros((LANES,), jnp.float32)
            return _
        lax.fori_loop(0, rpt // LANES, zb, None)

        def ob(i, _):
            obuf[pl.ds(i * LANES, LANES)] = jnp.ones((LANES,), jnp.float32)
            return _
        lax.fori_loop(0, K // LANES, ob, None)

        pltpu.sync_copy(zbuf, deg_sh.at[pl.ds(s * rpt, rpt)])

        @pl.when(c == 0)
        def _():
            pltpu.sync_copy(ei_hbm.at[1, pl.ds(s * cha, cha)],
                            idx_d.at[pl.ds(0, cha)])

        @pl.when(c == 1)
        def _():
            pltpu.sync_copy(ei_hbm.at[1, pl.ds(NS * cha + s * chb, chb)],
                            idx_d.at[pl.ds(0, chb)])

        plsc.subcore_barrier()

        def step(j, _):
            pltpu.sync_copy(obuf, deg_sh.at[idx_d.at[j]], add=True)
            return _

        @pl.when(c == 0)
        def _():
            lax.fori_loop(0, cha, step, None)

        @pl.when(c == 1)
        def _():
            lax.fori_loop(0, chb, step, None)

        plsc.subcore_barrier()
        pltpu.sync_copy(deg_sh.at[pl.ds(s * rpt, rpt)],
                        out_hbm.at[c, pl.ds(s * rpt, rpt)])

    return pl.kernel(
        body,
        out_type=jax.ShapeDtypeStruct((NC, n_pad), jnp.float32),
        mesh=_sc_mesh(),
        scratch_types=[
            pltpu.VMEM((chm, K), jnp.int32),
            pltpu.VMEM((rpt,), jnp.float32),
            pltpu.VMEM((K,), jnp.float32),
            pltpu.VMEM_SHARED((n_pad,), jnp.float32),
        ],
        compiler_params=pltpu.CompilerParams(use_tc_tiling_on_sc=False),
    )


def _segsum_kernel(n_pad, d, cha, chb):
    rpt = n_pad // NS
    chm = max(cha, chb)

    def body(hs_hbm, ei_hbm, out_hbm, idx_s, idx_d, r0, r1,
             zbuf, agg_sh, hs_sh, g0, g1, c0, c1):
        rows = [r0, r1]
        gsem = [g0, g1]
        csem = [c0, c1]
        c = lax.axis_index("c")
        s = lax.axis_index("s")

        pltpu.sync_copy(hs_hbm.at[pl.ds(s * rpt, rpt)],
                        hs_sh.at[pl.ds(s * rpt, rpt)])

        def zb(i, _):
            zbuf[i // (d // LANES), pl.ds((i % (d // LANES)) * LANES, LANES)] = (
                jnp.zeros((LANES,), jnp.float32))
            return _
        lax.fori_loop(0, ZR * d // LANES, zb, None)

        def zs(r, _):
            pltpu.sync_copy(zbuf, agg_sh.at[pl.ds(s * rpt + r * ZR, ZR)])
            return _
        lax.fori_loop(0, rpt // ZR, zs, None)

        @pl.when(c == 0)
        def _():
            pltpu.sync_copy(ei_hbm.at[0, pl.ds(s * cha, cha)],
                            idx_s.at[pl.ds(0, cha)])
            pltpu.sync_copy(ei_hbm.at[1, pl.ds(s * cha, cha)],
                            idx_d.at[pl.ds(0, cha)])

        @pl.when(c == 1)
        def _():
            pltpu.sync_copy(ei_hbm.at[0, pl.ds(NS * cha + s * chb, chb)],
                            idx_s.at[pl.ds(0, chb)])
            pltpu.sync_copy(ei_hbm.at[1, pl.ds(NS * cha + s * chb, chb)],
                            idx_d.at[pl.ds(0, chb)])

        plsc.subcore_barrier()

        def run_edges(chc):
            pltpu.async_copy(hs_sh.at[idx_s.at[0]], rows[0], gsem[0])

            def step(j2, _):
                j = j2 * 2
                pltpu.async_copy(hs_sh.at[idx_s.at[j + 1]], rows[1], gsem[1])
                pltpu.make_async_copy(hs_sh.at[idx_s.at[j]], rows[0],
                                      gsem[0]).wait()
                pltpu.sync_copy(rows[0], agg_sh.at[idx_d.at[j]], add=True)
                jn = jnp.minimum(j + 2, chc - 1)
                pltpu.async_copy(hs_sh.at[idx_s.at[jn]], rows[0], gsem[0])
                pltpu.make_async_copy(hs_sh.at[idx_s.at[j + 1]], rows[1],
                                      gsem[1]).wait()
                pltpu.sync_copy(rows[1], agg_sh.at[idx_d.at[j + 1]], add=True)
                return _
            lax.fori_loop(0, chc // 2, step, None)
            pltpu.make_async_copy(hs_sh.at[idx_s.at[0]], rows[0],
                                  gsem[0]).wait()

        @pl.when(c == 0)
        def _():
            run_edges(cha)

        @pl.when(c == 1)
        def _():
            run_edges(chb)

        plsc.subcore_barrier()
        pltpu.sync_copy(agg_sh.at[pl.ds(s * rpt, rpt)],
                        out_hbm.at[c, pl.ds(s * rpt, rpt)])

    return pl.kernel(
        body,
        out_type=jax.ShapeDtypeStruct((NC, n_pad, d), jnp.float32),
        mesh=_sc_mesh(),
        scratch_types=[
            pltpu.VMEM((chm, K), jnp.int32),
            pltpu.VMEM((chm, K), jnp.int32),
            pltpu.VMEM((K, d), jnp.float32),
            pltpu.VMEM((K, d), jnp.float32),
            pltpu.VMEM((ZR, d), jnp.float32),
            pltpu.VMEM_SHARED((n_pad, d), jnp.float32),
            pltpu.VMEM_SHARED((n_pad, d), jnp.float32),
        ] + [pltpu.SemaphoreType.DMA] * 4,
        compiler_params=pltpu.CompilerParams(use_tc_tiling_on_sc=False),
    )


def _tc1(n, n_pad, f, h):
    def body(degT_ref, x_ref, w_ref, b_ref, hs_ref, isq_ref):
        i = pl.program_id(0)
        deg = degT_ref[:, 0:1] + degT_ref[:, 1:2] + 1.0
        isq = lax.rsqrt(deg)
        rows = i * BLK + lax.broadcasted_iota(jnp.int32, (BLK, 1), 0)
        isq = jnp.where(rows < n, isq, 0.0)
        hw = jnp.dot(x_ref[...], w_ref[...],
                     preferred_element_type=jnp.float32) + b_ref[...]
        hs_ref[...] = hw * isq
        isq_ref[...] = isq

    grid = n_pad // BLK
    return pl.pallas_call(
        body,
        grid=(grid,),
        in_specs=[
            pl.BlockSpec((BLK, 2), lambda i: (i, 0)),
            pl.BlockSpec((BLK, f), lambda i: (i, 0)),
            pl.BlockSpec((f, h), lambda i: (0, 0)),
            pl.BlockSpec((1, h), lambda i: (0, 0)),
        ],
        out_specs=[
            pl.BlockSpec((BLK, h), lambda i: (i, 0)),
            pl.BlockSpec((BLK, 1), lambda i: (i, 0)),
        ],
        out_shape=[
            jax.ShapeDtypeStruct((n_pad, h), jnp.float32),
            jax.ShapeDtypeStruct((n_pad, 1), jnp.float32),
        ],
    )


def _tc2(n_pad, h, d2, c_out):
    def body(s1_ref, hs1_ref, isq_ref, w_ref, b_ref, hs2_ref):
        isq = isq_ref[...]
        ssum = s1_ref[0] + s1_ref[1] + hs1_ref[...]
        embed = jnp.maximum(isq * ssum, 0.0)
        hw = jnp.dot(embed, w_ref[...],
                     preferred_element_type=jnp.float32) + b_ref[...]
        hs2_ref[:, c_out:] = jnp.zeros((BLK, d2 - c_out), jnp.float32)
        hs2_ref[:, :c_out] = hw * isq

    grid = n_pad // BLK
    return pl.pallas_call(
        body,
        grid=(grid,),
        in_specs=[
            pl.BlockSpec((NC, BLK, h), lambda i: (0, i, 0)),
            pl.BlockSpec((BLK, h), lambda i: (i, 0)),
            pl.BlockSpec((BLK, 1), lambda i: (i, 0)),
            pl.BlockSpec((h, c_out), lambda i: (0, 0)),
            pl.BlockSpec((1, c_out), lambda i: (0, 0)),
        ],
        out_specs=pl.BlockSpec((BLK, d2), lambda i: (i, 0)),
        out_shape=jax.ShapeDtypeStruct((n_pad, d2), jnp.float32),
    )


def _tc3(n, n_pad, d2, c_out):
    def body(s2_ref, hs2_ref, isq_ref, out_ref):
        out = isq_ref[...] * (s2_ref[0] + s2_ref[1] + hs2_ref[...])
        out_ref[...] = out[:, :c_out]

    grid = n_pad // BLK
    return pl.pallas_call(
        body,
        grid=(grid,),
        in_specs=[
            pl.BlockSpec((NC, BLK, d2), lambda i: (0, i, 0)),
            pl.BlockSpec((BLK, d2), lambda i: (i, 0)),
            pl.BlockSpec((BLK, 1), lambda i: (i, 0)),
        ],
        out_specs=pl.BlockSpec((BLK, c_out), lambda i: (i, 0)),
        out_shape=jax.ShapeDtypeStruct((n, c_out), jnp.float32),
    )


def kernel(x, y, edge_index, W1, b1, W2, b2):
    n, f = x.shape
    h = W1.shape[1]
    c_out = W2.shape[1]
    d2 = 16

    src = edge_index[0]
    dst = edge_index[1]
    e = src.shape[0]
    total_ch = -(-(-(-e // (NS * K))) // 16) * 16
    e_pad = NS * K * total_ch
    n_pad = -(-(n + 1) // (NS * ZR)) * (NS * ZR)

    pad_e = e_pad - e
    eip = jnp.concatenate(
        [edge_index, jnp.full((2, pad_e), n, jnp.int32)],
        axis=1).reshape(2, NS * total_ch, K)
    b1r = b1.reshape(1, h)
    b2r = b2.reshape(1, c_out)

    deg_parts = _deg_kernel(n_pad, *_split(total_ch, FRAC_DEG))(eip)
    hs1, isq = _tc1(n, n_pad, f, h)(deg_parts.T, x, W1, b1r)
    s1 = _segsum_kernel(n_pad, h, *_split(total_ch, FRAC_S64))(hs1, eip)
    hs2 = _tc2(n_pad, h, d2, c_out)(s1, hs1, isq, W2, b2r)
    s2 = _segsum_kernel(n_pad, d2, *_split(total_ch, FRAC_S16))(hs2, eip)
    return _tc3(n, n_pad, d2, c_out)(s2, hs2, isq)


# speedup vs baseline: 1.1079x; 1.0171x over previous
import functools
import jax
import jax.numpy as jnp
from jax import lax
from jax.experimental import pallas as pl
from jax.experimental.pallas import tpu as pltpu
from jax.experimental.pallas import tpu_sc as plsc

NC, NS, LANES = 2, 16, 16
NW = NC * NS
K = 128
ZR = 64
BLK = 10240

FRAC_DEG = 0.55
FRAC_S64 = 0.51
FRAC_S16 = 0.56


def _split(total, frac):
    a = int(round(total * frac / 8.0)) * 8
    a = max(8, min(total - 8, a))
    return a, total - a


def _sc_mesh():
    return plsc.VectorSubcoreMesh(core_axis_name="c", subcore_axis_name="s")


def _deg_kernel(n_pad, cha, chb):
    rpt = n_pad // NS
    chm = max(cha, chb)

    def body(ei_hbm, out_hbm, idx_d, zbuf, obuf, deg_sh):
        c = lax.axis_index("c")
        s = lax.axis_index("s")

        def zb(i, _):
            zbuf[pl.ds(i * LANES, LANES)] = jnp.zeros((LANES,), jnp.float32)
            return _
        lax.fori_loop(0, rpt // LANES, zb, None)

        def ob(i, _):
            obuf[pl.ds(i * LANES, LANES)] = jnp.ones((LANES,), jnp.float32)
            return _
        lax.fori_loop(0, K // LANES, ob, None)

        pltpu.sync_copy(zbuf, deg_sh.at[pl.ds(s * rpt, rpt)])

        @pl.when(c == 0)
        def _():
            pltpu.sync_copy(ei_hbm.at[1, pl.ds(s * cha, cha)],
                            idx_d.at[pl.ds(0, cha)])

        @pl.when(c == 1)
        def _():
            pltpu.sync_copy(ei_hbm.at[1, pl.ds(NS * cha + s * chb, chb)],
                            idx_d.at[pl.ds(0, chb)])

        plsc.subcore_barrier()

        def step(j, _):
            pltpu.sync_copy(obuf, deg_sh.at[idx_d.at[j]], add=True)
            return _

        @pl.when(c == 0)
        def _():
            lax.fori_loop(0, cha, step, None)

        @pl.when(c == 1)
        def _():
            lax.fori_loop(0, chb, step, None)

        plsc.subcore_barrier()
        pltpu.sync_copy(deg_sh.at[pl.ds(s * rpt, rpt)],
                        out_hbm.at[c, pl.ds(s * rpt, rpt)])

    return pl.kernel(
        body,
        out_type=jax.ShapeDtypeStruct((NC, n_pad), jnp.float32),
        mesh=_sc_mesh(),
        scratch_types=[
            pltpu.VMEM((chm, K), jnp.int32),
            pltpu.VMEM((rpt,), jnp.float32),
            pltpu.VMEM((K,), jnp.float32),
            pltpu.VMEM_SHARED((n_pad,), jnp.float32),
        ],
        compiler_params=pltpu.CompilerParams(use_tc_tiling_on_sc=False),
    )


def _segsum_kernel(n_pad, d, cha, chb):
    rpt = n_pad // NS
    chm = max(cha, chb)

    def body(hs_hbm, ei_hbm, out_hbm, idx_s, idx_d, r0, r1,
             zbuf, agg_sh, hs_sh, g0, g1, c0, c1):
        rows = [r0, r1]
        gsem = [g0, g1]
        csem = [c0, c1]
        c = lax.axis_index("c")
        s = lax.axis_index("s")

        pltpu.sync_copy(hs_hbm.at[pl.ds(s * rpt, rpt)],
                        hs_sh.at[pl.ds(s * rpt, rpt)])

        def zb(i, _):
            zbuf[i // (d // LANES), pl.ds((i % (d // LANES)) * LANES, LANES)] = (
                jnp.zeros((LANES,), jnp.float32))
            return _
        lax.fori_loop(0, ZR * d // LANES, zb, None)

        def zs(r, _):
            pltpu.sync_copy(zbuf, agg_sh.at[pl.ds(s * rpt + r * ZR, ZR)])
            return _
        lax.fori_loop(0, rpt // ZR, zs, None)

        @pl.when(c == 0)
        def _():
            pltpu.sync_copy(ei_hbm.at[0, pl.ds(s * cha, cha)],
                            idx_s.at[pl.ds(0, cha)])
            pltpu.sync_copy(ei_hbm.at[1, pl.ds(s * cha, cha)],
                            idx_d.at[pl.ds(0, cha)])

        @pl.when(c == 1)
        def _():
            pltpu.sync_copy(ei_hbm.at[0, pl.ds(NS * cha + s * chb, chb)],
                            idx_s.at[pl.ds(0, chb)])
            pltpu.sync_copy(ei_hbm.at[1, pl.ds(NS * cha + s * chb, chb)],
                            idx_d.at[pl.ds(0, chb)])

        plsc.subcore_barrier()

        def run_edges(chc):
            pltpu.async_copy(hs_sh.at[idx_s.at[0]], rows[0], gsem[0])

            def step(j2, _):
                j = j2 * 2
                pltpu.async_copy(hs_sh.at[idx_s.at[j + 1]], rows[1], gsem[1])
                pltpu.make_async_copy(hs_sh.at[idx_s.at[j]], rows[0],
                                      gsem[0]).wait()
                pltpu.sync_copy(rows[0], agg_sh.at[idx_d.at[j]], add=True)
                jn = jnp.minimum(j + 2, chc - 1)
                pltpu.async_copy(hs_sh.at[idx_s.at[jn]], rows[0], gsem[0])
                pltpu.make_async_copy(hs_sh.at[idx_s.at[j + 1]], rows[1],
                                      gsem[1]).wait()
                pltpu.sync_copy(rows[1], agg_sh.at[idx_d.at[j + 1]], add=True)
                return _
            lax.fori_loop(0, chc // 2, step, None)
            pltpu.make_async_copy(hs_sh.at[idx_s.at[0]], rows[0],
                                  gsem[0]).wait()

        @pl.when(c == 0)
        def _():
            run_edges(cha)

        @pl.when(c == 1)
        def _():
            run_edges(chb)

        plsc.subcore_barrier()
        pltpu.sync_copy(agg_sh.at[pl.ds(s * rpt, rpt)],
                        out_hbm.at[c, pl.ds(s * rpt, rpt)])

    return pl.kernel(
        body,
        out_type=jax.ShapeDtypeStruct((NC, n_pad, d), jnp.float32),
        mesh=_sc_mesh(),
        scratch_types=[
            pltpu.VMEM((chm, K), jnp.int32),
            pltpu.VMEM((chm, K), jnp.int32),
            pltpu.VMEM((K, d), jnp.float32),
            pltpu.VMEM((K, d), jnp.float32),
            pltpu.VMEM((ZR, d), jnp.float32),
            pltpu.VMEM_SHARED((n_pad, d), jnp.float32),
            pltpu.VMEM_SHARED((n_pad, d), jnp.float32),
        ] + [pltpu.SemaphoreType.DMA] * 4,
        compiler_params=pltpu.CompilerParams(use_tc_tiling_on_sc=False),
    )


def _tc1(n, n_pad, f, h):
    def body(degT_ref, x_ref, w_ref, b_ref, hs_ref, isq_ref):
        i = pl.program_id(0)
        deg = degT_ref[:, 0:1] + degT_ref[:, 1:2] + 1.0
        isq = lax.rsqrt(deg)
        rows = i * BLK + lax.broadcasted_iota(jnp.int32, (BLK, 1), 0)
        isq = jnp.where(rows < n, isq, 0.0)
        hw = jnp.dot(x_ref[...], w_ref[...],
                     preferred_element_type=jnp.float32) + b_ref[...]
        hs_ref[...] = hw * isq
        isq_ref[...] = isq

    grid = n_pad // BLK
    return pl.pallas_call(
        body,
        grid=(grid,),
        in_specs=[
            pl.BlockSpec((BLK, 2), lambda i: (i, 0)),
            pl.BlockSpec((BLK, f), lambda i: (i, 0)),
            pl.BlockSpec((f, h), lambda i: (0, 0)),
            pl.BlockSpec((1, h), lambda i: (0, 0)),
        ],
        out_specs=[
            pl.BlockSpec((BLK, h), lambda i: (i, 0)),
            pl.BlockSpec((BLK, 1), lambda i: (i, 0)),
        ],
        out_shape=[
            jax.ShapeDtypeStruct((n_pad, h), jnp.float32),
            jax.ShapeDtypeStruct((n_pad, 1), jnp.float32),
        ],
    )


def _tc2(n_pad, h, d2, c_out):
    def body(s1_ref, hs1_ref, isq_ref, w_ref, b_ref, hs2_ref):
        isq = isq_ref[...]
        ssum = s1_ref[0] + s1_ref[1] + hs1_ref[...]
        embed = jnp.maximum(isq * ssum, 0.0)
        hw = jnp.dot(embed, w_ref[...],
                     preferred_element_type=jnp.float32) + b_ref[...]
        hs2_ref[:, c_out:] = jnp.zeros((BLK, d2 - c_out), jnp.float32)
        hs2_ref[:, :c_out] = hw * isq

    grid = n_pad // BLK
    return pl.pallas_call(
        body,
        grid=(grid,),
        in_specs=[
            pl.BlockSpec((NC, BLK, h), lambda i: (0, i, 0)),
            pl.BlockSpec((BLK, h), lambda i: (i, 0)),
            pl.BlockSpec((BLK, 1), lambda i: (i, 0)),
            pl.BlockSpec((h, c_out), lambda i: (0, 0)),
            pl.BlockSpec((1, c_out), lambda i: (0, 0)),
        ],
        out_specs=pl.BlockSpec((BLK, d2), lambda i: (i, 0)),
        out_shape=jax.ShapeDtypeStruct((n_pad, d2), jnp.float32),
    )


def _tc3(n, n_pad, d2, c_out):
    def body(s2_ref, hs2_ref, isq_ref, out_ref):
        out = isq_ref[...] * (s2_ref[0] + s2_ref[1] + hs2_ref[...])
        out_ref[...] = out[:, :c_out]

    grid = n_pad // BLK
    return pl.pallas_call(
        body,
        grid=(grid,),
        in_specs=[
            pl.BlockSpec((NC, BLK, d2), lambda i: (0, i, 0)),
            pl.BlockSpec((BLK, d2), lambda i: (i, 0)),
            pl.BlockSpec((BLK, 1), lambda i: (i, 0)),
        ],
        out_specs=pl.BlockSpec((BLK, c_out), lambda i: (i, 0)),
        out_shape=jax.ShapeDtypeStruct((n, c_out), jnp.float32),
    )


def kernel(x, y, edge_index, W1, b1, W2, b2):
    n, f = x.shape
    h = W1.shape[1]
    c_out = W2.shape[1]
    d2 = 16

    src = edge_index[0]
    dst = edge_index[1]
    e = src.shape[0]
    total_ch = -(-(-(-e // (NS * K))) // 16) * 16
    e_pad = NS * K * total_ch
    n_pad = -(-(n + 1) // (NS * ZR)) * (NS * ZR)

    pad_e = e_pad - e
    eip = jnp.concatenate(
        [edge_index, jnp.full((2, pad_e), n, jnp.int32)],
        axis=1).reshape(2, NS * total_ch, K)
    b1r = b1.reshape(1, h)
    b2r = b2.reshape(1, c_out)

    deg_parts = _deg_kernel(n_pad, *_split(total_ch, FRAC_DEG))(eip)
    hs1, isq = _tc1(n, n_pad, f, h)(deg_parts.T, x, W1, b1r)
    s1 = _segsum_kernel(n_pad, h, *_split(total_ch, FRAC_S64))(hs1, eip)
    hs2 = _tc2(n_pad, h, d2, c_out)(s1, hs1, isq, W2, b2r)
    s2 = _segsum_kernel(n_pad, d2, *_split(total_ch, FRAC_S16))(hs2, eip)
    return _tc3(n, n_pad, d2, c_out)(s2, hs2, isq)
